# baseline (speedup 1.0000x reference)
import functools

import jax
import jax.numpy as jnp
from jax import lax
from jax.experimental import pallas as pl
from jax.experimental.pallas import tpu as pltpu
from jax.experimental.pallas import tpu_sc as plsc


def _sc_gather(table, idx):
    v, d = table.shape
    b = idx.shape[0]
    info = plsc.get_sparse_core_info()
    nc, ns = info.num_cores, info.num_subcores
    nw = nc * ns
    assert b % nw == 0 and (b // nw) % 8 == 0
    b_per_w = b // nw
    mesh = plsc.VectorSubcoreMesh(core_axis_name="c", subcore_axis_name="s")

    @functools.partial(
        pl.kernel,
        mesh=mesh,
        out_type=jax.ShapeDtypeStruct((b, d), jnp.float32),
        compiler_params=pltpu.CompilerParams(use_tc_tiling_on_sc=False),
        scratch_types=[
            pltpu.VMEM((b_per_w,), jnp.int32),
            pltpu.VMEM((b_per_w, d), jnp.float32),
            pltpu.SemaphoreType.DMA,
        ],
    )
    def gather_kernel(table_hbm, idx_hbm, out_hbm, idx_v, rows_v, sem):
        wid = lax.axis_index("s") * nc + lax.axis_index("c")
        base = wid * b_per_w
        pltpu.sync_copy(idx_hbm.at[pl.ds(base, b_per_w)], idx_v)
        pltpu.async_copy(table_hbm.at[idx_v], rows_v, sem).wait()
        pltpu.sync_copy(rows_v, out_hbm.at[pl.ds(base, b_per_w)])

    return gather_kernel(table, idx)


def _mlp(e, w_h, b_h2, w_o, b_o2, vt=2048):
    bsz, wd = e.shape
    h = w_h.shape[0]
    v = w_o.shape[0]
    grid = (v + vt - 1) // vt

    def body(e_ref, wh_ref, bh_ref, wo_ref, bo_ref, out_ref, h_ref):
        @pl.when(pl.program_id(0) == 0)
        def _():
            acc = lax.dot_general(
                e_ref[...], wh_ref[...], (((1,), (1,)), ((), ())),
                preferred_element_type=jnp.float32)
            h_ref[...] = jnp.tanh(acc + bh_ref[...])

        out_ref[...] = lax.dot_general(
            h_ref[...], wo_ref[...], (((1,), (1,)), ((), ())),
            preferred_element_type=jnp.float32) + bo_ref[...]

    return pl.pallas_call(
        body,
        grid=(grid,),
        in_specs=[
            pl.BlockSpec((bsz, wd), lambda j: (0, 0)),
            pl.BlockSpec((h, wd), lambda j: (0, 0)),
            pl.BlockSpec((1, h), lambda j: (0, 0)),
            pl.BlockSpec((vt, h), lambda j: (j, 0)),
            pl.BlockSpec((1, vt), lambda j: (0, j)),
        ],
        out_specs=pl.BlockSpec((bsz, vt), lambda j: (0, j)),
        out_shape=jax.ShapeDtypeStruct((bsz, v), jnp.float32),
        scratch_shapes=[pltpu.VMEM((bsz, h), jnp.float32)],
    )(e, w_h, b_h2, w_o, b_o2)


def kernel(x, emb, W_h, b_h, W_o, b_o):
    batch, window = x.shape
    dim = emb.shape[1]
    idx = x.reshape(-1).astype(jnp.int32)
    rows = _sc_gather(emb, idx)
    e = rows.reshape(batch, window * dim)
    return _mlp(e, W_h, b_h.reshape(1, -1), W_o, b_o.reshape(1, -1))


# speedup vs baseline: 1.0026x; 1.0026x over previous
import functools

import jax
import jax.numpy as jnp
from jax import lax
from jax.experimental import pallas as pl
from jax.experimental.pallas import tpu as pltpu
from jax.experimental.pallas import tpu_sc as plsc


def _sc_gather(table, idx):
    v, d = table.shape
    b = idx.shape[0]
    info = plsc.get_sparse_core_info()
    nc, ns = info.num_cores, info.num_subcores
    nw = nc * ns
    assert b % nw == 0 and (b // nw) % 8 == 0
    b_per_w = b // nw
    mesh = plsc.VectorSubcoreMesh(core_axis_name="c", subcore_axis_name="s")

    @functools.partial(
        pl.kernel,
        mesh=mesh,
        out_type=jax.ShapeDtypeStruct((b, d), jnp.float32),
        compiler_params=pltpu.CompilerParams(use_tc_tiling_on_sc=False),
        scratch_types=[
            pltpu.VMEM((b_per_w,), jnp.int32),
            pltpu.VMEM((b_per_w, d), jnp.float32),
            pltpu.SemaphoreType.DMA,
        ],
    )
    def gather_kernel(table_hbm, idx_hbm, out_hbm, idx_v, rows_v, sem):
        wid = lax.axis_index("s") * nc + lax.axis_index("c")
        base = wid * b_per_w
        pltpu.sync_copy(idx_hbm.at[pl.ds(base, b_per_w)], idx_v)
        pltpu.async_copy(table_hbm.at[idx_v], rows_v, sem).wait()
        pltpu.sync_copy(rows_v, out_hbm.at[pl.ds(base, b_per_w)])

    return gather_kernel(table, idx)


def _mlp(e, w_h, b_h2, w_o, b_o2, vt=4096):
    bsz, wd = e.shape
    h = w_h.shape[0]
    v = w_o.shape[0]
    grid = (v + vt - 1) // vt

    def body(e_ref, wh_ref, bh_ref, wo_ref, bo_ref, out_ref, h_ref):
        @pl.when(pl.program_id(0) == 0)
        def _():
            acc = lax.dot_general(
                e_ref[...], wh_ref[...], (((1,), (1,)), ((), ())),
                preferred_element_type=jnp.float32)
            h_ref[...] = jnp.tanh(acc + bh_ref[...])

        out_ref[...] = lax.dot_general(
            h_ref[...], wo_ref[...], (((1,), (1,)), ((), ())),
            preferred_element_type=jnp.float32) + bo_ref[...]

    return pl.pallas_call(
        body,
        grid=(grid,),
        in_specs=[
            pl.BlockSpec((bsz, wd), lambda j: (0, 0)),
            pl.BlockSpec((h, wd), lambda j: (0, 0)),
            pl.BlockSpec((1, h), lambda j: (0, 0)),
            pl.BlockSpec((vt, h), lambda j: (j, 0)),
            pl.BlockSpec((1, vt), lambda j: (0, j)),
        ],
        out_specs=pl.BlockSpec((bsz, vt), lambda j: (0, j)),
        out_shape=jax.ShapeDtypeStruct((bsz, v), jnp.float32),
        scratch_shapes=[pltpu.VMEM((bsz, h), jnp.float32)],
    )(e, w_h, b_h2, w_o, b_o2)


def kernel(x, emb, W_h, b_h, W_o, b_o):
    batch, window = x.shape
    dim = emb.shape[1]
    idx = x.reshape(-1).astype(jnp.int32)
    rows = _sc_gather(emb, idx)
    e = rows.reshape(batch, window * dim)
    return _mlp(e, W_h, b_h.reshape(1, -1), W_o, b_o.reshape(1, -1))


# speedup vs baseline: 1.0673x; 1.0645x over previous
import functools

import jax
import jax.numpy as jnp
from jax import lax
from jax.experimental import pallas as pl
from jax.experimental.pallas import tpu as pltpu
from jax.experimental.pallas import tpu_sc as plsc


def _sc_gather(table, idx):
    v, d = table.shape
    b = idx.shape[0]
    info = plsc.get_sparse_core_info()
    nc, ns = info.num_cores, info.num_subcores
    nw = nc * ns
    assert b % nw == 0 and (b // nw) % 8 == 0
    b_per_w = b // nw
    mesh = plsc.VectorSubcoreMesh(core_axis_name="c", subcore_axis_name="s")

    @functools.partial(
        pl.kernel,
        mesh=mesh,
        out_type=jax.ShapeDtypeStruct((b, d), jnp.float32),
        compiler_params=pltpu.CompilerParams(use_tc_tiling_on_sc=False),
        scratch_types=[
            pltpu.VMEM((b_per_w,), jnp.int32),
            pltpu.VMEM((b_per_w, d), jnp.float32),
            pltpu.SemaphoreType.DMA,
        ],
    )
    def gather_kernel(table_hbm, idx_hbm, out_hbm, idx_v, rows_v, sem):
        wid = lax.axis_index("s") * nc + lax.axis_index("c")
        base = wid * b_per_w
        pltpu.sync_copy(idx_hbm.at[pl.ds(base, b_per_w)], idx_v)
        pltpu.async_copy(table_hbm.at[idx_v], rows_v, sem).wait()
        pltpu.sync_copy(rows_v, out_hbm.at[pl.ds(base, b_per_w)])

    return gather_kernel(table, idx)


def _mlp(e, w_h, b_h2, w_o, b_o2, vt=4096):
    bsz, wd = e.shape
    h = w_h.shape[0]
    v = w_o.shape[0]
    grid = (v + vt - 1) // vt

    def body(e_ref, wh_ref, bh_ref, wo_ref, bo_ref, out_ref, h_ref):
        @pl.when(pl.program_id(0) == 0)
        def _():
            acc = lax.dot_general(
                e_ref[...], wh_ref[...], (((1,), (1,)), ((), ())),
                preferred_element_type=jnp.float32)
            h_ref[...] = jnp.tanh(acc + bh_ref[...])

        out_ref[...] = lax.dot_general(
            h_ref[...], wo_ref[...], (((1,), (1,)), ((), ())),
            preferred_element_type=jnp.float32) + bo_ref[...]

    return pl.pallas_call(
        body,
        grid=(grid,),
        in_specs=[
            pl.BlockSpec((bsz, wd), lambda j: (0, 0)),
            pl.BlockSpec((h, wd), lambda j: (0, 0)),
            pl.BlockSpec((1, h), lambda j: (0, 0)),
            pl.BlockSpec((vt, h), lambda j: (j, 0)),
            pl.BlockSpec((1, vt), lambda j: (0, j)),
        ],
        out_specs=pl.BlockSpec((bsz, vt), lambda j: (0, j)),
        out_shape=jax.ShapeDtypeStruct((bsz, v), jnp.float32),
        scratch_shapes=[pltpu.VMEM((bsz, h), jnp.float32)],
    )(e, w_h, b_h2, w_o, b_o2)


def kernel(x, emb, W_h, b_h, W_o, b_o):
    batch, window = x.shape
    dim = emb.shape[1]
    idx = x.reshape(-1).astype(jnp.int32)
    rows = jnp.take(emb, idx, axis=0)
    e = rows.reshape(batch, window * dim)
    return _mlp(e, W_h, b_h.reshape(1, -1), W_o, b_o.reshape(1, -1))
